# baseline (speedup 1.0000x reference)
import functools

import jax
import jax.numpy as jnp
from jax import lax
from jax.experimental import pallas as pl
from jax.experimental.pallas import tpu as pltpu
from jax.experimental.pallas import tpu_sc as plsc

L = 2048
C = 16
P = 2 * L - 1
PAD0 = 8
QQ = 4104
NC = 2
NS = 16
NW = NC * NS
ROWS = L // NW
CT = C // 8
JT = L // 128
FW = 2568
VW = 2560


def _build():
    mesh = plsc.VectorSubcoreMesh(core_axis_name="c", subcore_axis_name="s")

    @functools.partial(
        pl.kernel,
        mesh=mesh,
        out_type=jax.ShapeDtypeStruct((L, CT, JT, 8, 128), jnp.float32),
        scratch_types=[
            pltpu.VMEM((C, FW), jnp.float32),
            pltpu.VMEM((C, VW), jnp.float32),
            pltpu.SemaphoreType.DMA,
        ],
        compiler_params=pltpu.CompilerParams(use_tc_tiling_on_sc=False),
    )
    def toeplitz_kernel(table_hbm, out_hbm, fw_v, v_v, sem):
        wid = lax.axis_index("s") * NC + lax.axis_index("c")
        d = wid & 7
        lo = (wid >> 3) * 512

        pltpu.sync_copy(table_hbm.at[:, pl.ds(lo, FW)], fw_v)

        def rev_body(n, carry):
            for u in range(4):
                k = n * 4 + u
                c = k // (VW // 16)
                m = k - c * (VW // 16)
                src = d + 2544 - 16 * m
                v_v[c, pl.ds(16 * m, 16)] = jnp.flip(fw_v[c, pl.ds(src, 16)])
            return carry

        lax.fori_loop(0, C * (VW // 16) // 4, rev_body, 0)

        def fire_plane(k):
            i = lo + d + 8 * k
            y0 = 504 - 8 * k
            for ct in range(CT):
                for jt in range(JT):
                    pltpu.async_copy(
                        v_v.at[pl.ds(ct * 8, 8), pl.ds(y0 + jt * 128, 128)],
                        out_hbm.at[i, ct, jt],
                        sem,
                    )

        def drain_plane():
            for ct in range(CT):
                for jt in range(JT):
                    pltpu.make_async_copy(
                        out_hbm.at[lo, ct, jt],
                        v_v.at[pl.ds(0, 8), pl.ds(0, 128)],
                        sem,
                    ).wait()

        fire_plane(0)

        def emit_plane(k, carry):
            fire_plane(k)
            drain_plane()
            return carry

        lax.fori_loop(1, ROWS, emit_plane, 0)
        drain_plane()

    return toeplitz_kernel


_KERNEL = _build()


def kernel(params, indices):
    del indices
    tab = jnp.concatenate(
        [
            jnp.zeros((C, PAD0), jnp.float32),
            params[0].T,
            jnp.zeros((C, QQ - PAD0 - P), jnp.float32),
        ],
        axis=1,
    )
    out5 = _KERNEL(tab)
    return out5.transpose(0, 2, 4, 1, 3).reshape(L, L, C)


# speedup vs baseline: 1.0050x; 1.0050x over previous
import functools

import jax
import jax.numpy as jnp
from jax import lax
from jax.experimental import pallas as pl
from jax.experimental.pallas import tpu as pltpu
from jax.experimental.pallas import tpu_sc as plsc

L = 2048
C = 16
P = 2 * L - 1
PAD0 = 8
QQ = 4104
NC = 2
NS = 16
NW = NC * NS
ROWS = L // NW
CT = C // 8
JT = L // 128
FW = 2568
VW = 2560


def _build():
    mesh = plsc.VectorSubcoreMesh(core_axis_name="c", subcore_axis_name="s")

    @functools.partial(
        pl.kernel,
        mesh=mesh,
        out_type=jax.ShapeDtypeStruct((L, CT, JT, 8, 128), jnp.float32),
        scratch_types=[
            pltpu.VMEM((C, FW), jnp.float32),
            pltpu.VMEM((C, VW), jnp.float32),
            pltpu.SemaphoreType.DMA,
        ],
        compiler_params=pltpu.CompilerParams(use_tc_tiling_on_sc=False),
    )
    def toeplitz_kernel(table_hbm, out_hbm, fw_v, v_v, sem):
        wid = lax.axis_index("s") * NC + lax.axis_index("c")
        d = wid & 7
        lo = (wid >> 3) * 512

        pltpu.sync_copy(table_hbm.at[:, pl.ds(lo, FW)], fw_v)

        def rev_body(n, carry):
            for u in range(4):
                k = n * 4 + u
                c = k // (VW // 16)
                m = k - c * (VW // 16)
                src = d + 2544 - 16 * m
                v_v[c, pl.ds(16 * m, 16)] = jnp.flip(fw_v[c, pl.ds(src, 16)])
            return carry

        lax.fori_loop(0, C * (VW // 16) // 4, rev_body, 0)

        def emit_plane(k, carry):
            i = lo + d + 8 * k
            y0 = 504 - 8 * k
            copies = []
            for ct in range(CT):
                for jt in range(JT):
                    copies.append(pltpu.async_copy(
                        v_v.at[pl.ds(ct * 8, 8), pl.ds(y0 + jt * 128, 128)],
                        out_hbm.at[i, ct, jt],
                        sem,
                    ))
            for cp in copies:
                cp.wait()
            return carry

        lax.fori_loop(0, ROWS, emit_plane, 0)

    return toeplitz_kernel


_KERNEL = _build()


def kernel(params, indices):
    del indices
    tab = jnp.concatenate(
        [
            jnp.zeros((C, PAD0), jnp.float32),
            params[0].T,
            jnp.zeros((C, QQ - PAD0 - P), jnp.float32),
        ],
        axis=1,
    )
    out5 = _KERNEL(tab)
    return out5.transpose(0, 2, 4, 1, 3).reshape(L, L, C)
